# baseline (speedup 1.0000x reference)
import functools

import jax
import jax.numpy as jnp
from jax import lax
from jax.experimental import pallas as pl
from jax.experimental.pallas import tpu as pltpu
from jax.experimental.pallas import tpu_sc as plsc

_ILV = plsc.PackFormat.INTERLEAVED

POOL = 7
H = 64
W = 64
C = 256
N_ROI = 2000
ROWS = N_ROI * POOL * POOL
GROUP = 16
NCHUNKS = ROWS // GROUP
NC, NS = 2, 16
NW = NC * NS
CHUNKS_PER = -(-NCHUNKS // NW)


def _rpn_pool_body(feat_hbm, roi_hbm, tab_hbm, out_hbm, roi_v, tab_v, idx0,
                   idx1, w0, w1, g0, g1, ob0, ob1, semg0, semg1, semo0,
                   semo1):
    idx_b = (idx0, idx1)
    w_b = (w0, w1)
    g_b = (g0, g1)
    ob_b = (ob0, ob1)
    semg = (semg0, semg1)
    semo = (semo0, semo1)

    wid = lax.axis_index("s") * NC + lax.axis_index("c")
    pltpu.sync_copy(roi_hbm, roi_v)
    start = wid * CHUNKS_PER
    count = jnp.minimum(CHUNKS_PER, NCHUNKS - start)
    lanes = lax.iota(jnp.int32, 16)
    l4 = lanes * 4

    pltpu.sync_copy(tab_hbm, tab_v)

    def issue_gather(k, b):
        p = (start + k) * GROUP + lanes
        n = p // (POOL * POOL)
        q = p - n * (POOL * POOL)
        i = plsc.load_gather(tab_v, [q])
        j = q - i * POOL
        b4 = n * 4
        y1 = plsc.load_gather(roi_v, [b4])
        x1 = plsc.load_gather(roi_v, [b4 + 1])
        y2 = plsc.load_gather(roi_v, [b4 + 2])
        x2 = plsc.load_gather(roi_v, [b4 + 3])
        h = jnp.maximum(x2 - x1, 1)
        w = jnp.maximum(y2 - y1, 1)
        h7 = h.astype(jnp.float32) / float(POOL)
        w7 = w.astype(jnp.float32) / float(POOL)
        rpos = i.astype(jnp.float32) * h7
        cpos = j.astype(jnp.float32) * w7
        r0 = rpos.astype(jnp.int32)
        c0 = cpos.astype(jnp.int32)
        rf = rpos - r0.astype(jnp.float32)
        cf = cpos - c0.astype(jnp.float32)
        r1 = jnp.minimum(r0 + 1, h - 1)
        c1 = jnp.minimum(c0 + 1, w - 1)
        R0 = jnp.clip(x1 + r0, 0, H - 1)
        R1 = jnp.clip(x1 + r1, 0, H - 1)
        C0 = jnp.clip(y1 + c0, 0, W - 1)
        C1 = jnp.clip(y1 + c1, 0, W - 1)
        plsc.store_scatter(idx_b[b], [l4], R0 * W + C0)
        plsc.store_scatter(idx_b[b], [l4 + 1], R0 * W + C1)
        plsc.store_scatter(idx_b[b], [l4 + 2], R1 * W + C0)
        plsc.store_scatter(idx_b[b], [l4 + 3], R1 * W + C1)
        wr0 = 1.0 - rf
        wc0 = 1.0 - cf
        plsc.store_scatter(w_b[b], [l4], wr0 * wc0)
        plsc.store_scatter(w_b[b], [l4 + 1], wr0 * cf)
        plsc.store_scatter(w_b[b], [l4 + 2], rf * wc0)
        plsc.store_scatter(w_b[b], [l4 + 3], rf * cf)
        pltpu.async_copy(feat_hbm.at[idx_b[b]], g_b[b], semg[b])

    def process(k, b):
        @pl.when(k + 1 < count)
        def _pref():
            issue_gather(k + 1, 1 - b)

        pltpu.make_async_copy(feat_hbm.at[idx_b[b]], g_b[b], semg[b]).wait()

        @pl.when(k >= 2)
        def _drain():
            pltpu.make_async_copy(
                ob_b[b],
                out_hbm.at[pl.ds((start + k - 2) * GROUP * C, GROUP * C)],
                semo[b]).wait()

        g_v = g_b[b]
        ob_v = ob_b[b]
        w_v = w_b[b]

        def row_body(r, rcarry):
            r4 = r * 4
            w00 = plsc.load_gather(w_v, [jnp.full((16,), 0, jnp.int32) + r4])
            w01 = plsc.load_gather(w_v, [jnp.full((16,), 1, jnp.int32) + r4])
            w10 = plsc.load_gather(w_v, [jnp.full((16,), 2, jnp.int32) + r4])
            w11 = plsc.load_gather(w_v, [jnp.full((16,), 3, jnp.int32) + r4])
            for t in range(C // 32):
                s = pl.ds(t * 16, 16)
                a0, b0 = plsc.unpack(
                    plsc.bitcast(g_v[r4, s], jnp.bfloat16), format=_ILV)
                a1, b1 = plsc.unpack(
                    plsc.bitcast(g_v[r4 + 1, s], jnp.bfloat16), format=_ILV)
                a2, b2 = plsc.unpack(
                    plsc.bitcast(g_v[r4 + 2, s], jnp.bfloat16), format=_ILV)
                a3, b3 = plsc.unpack(
                    plsc.bitcast(g_v[r4 + 3, s], jnp.bfloat16), format=_ILV)
                acca = (a0 * w00 + a1 * w01) + (a2 * w10 + a3 * w11)
                accb = (b0 * w00 + b1 * w01) + (b2 * w10 + b3 * w11)
                ob_v[pl.ds(r * C + t * 16, 16)] = acca
                ob_v[pl.ds(r * C + C // 2 + t * 16, 16)] = accb
            return rcarry

        lax.fori_loop(0, GROUP, row_body, 0, unroll=4)
        pltpu.async_copy(
            ob_v, out_hbm.at[pl.ds((start + k) * GROUP * C, GROUP * C)],
            semo[b])

    issue_gather(0, 0)

    def outer(t, carry):
        for b in range(2):
            k = t * 2 + b

            @pl.when(k < count)
            def _():
                process(k, b)

        return carry

    lax.fori_loop(0, CHUNKS_PER // 2, outer, 0)

    for b in range(2):
        last = count - 1 - ((count - 1 + b) % 2)
        pltpu.make_async_copy(
            ob_b[b], out_hbm.at[pl.ds((start + last) * GROUP * C, GROUP * C)],
            semo[b]).wait()


@functools.partial(
    pl.kernel,
    out_type=jax.ShapeDtypeStruct((ROWS * C,), jnp.float32),
    mesh=plsc.VectorSubcoreMesh(core_axis_name="c", subcore_axis_name="s"),
    scratch_types=[
        pltpu.VMEM((N_ROI * 4,), jnp.int32),
        pltpu.VMEM((64,), jnp.int32),
        pltpu.VMEM((GROUP * 4,), jnp.int32),
        pltpu.VMEM((GROUP * 4,), jnp.int32),
        pltpu.VMEM((GROUP * 4,), jnp.float32),
        pltpu.VMEM((GROUP * 4,), jnp.float32),
        pltpu.VMEM((GROUP * 4, C // 2), jnp.int32),
        pltpu.VMEM((GROUP * 4, C // 2), jnp.int32),
        pltpu.VMEM((GROUP * C,), jnp.float32),
        pltpu.VMEM((GROUP * C,), jnp.float32),
        pltpu.SemaphoreType.DMA,
        pltpu.SemaphoreType.DMA,
        pltpu.SemaphoreType.DMA,
        pltpu.SemaphoreType.DMA,
    ],
    compiler_params=pltpu.CompilerParams(needs_layout_passes=False),
)
def _rpn_pool_sc(feat_hbm, roi_hbm, tab_hbm, out_hbm, roi_v, tab_v, idx0,
                 idx1, w0, w1, g0, g1, ob0, ob1, semg0, semg1, semo0, semo1):
    _rpn_pool_body(feat_hbm, roi_hbm, tab_hbm, out_hbm, roi_v, tab_v, idx0,
                   idx1, w0, w1, g0, g1, ob0, ob1, semg0, semg1, semo0, semo1)


_QDIV_TAB = [min(e, POOL * POOL - 1) // POOL for e in range(64)]


_COL_PERM = [(e // 2) + (e % 2) * (C // 2) for e in range(C)]


def kernel(features, roi):
    feat = features.reshape(H * W, C).astype(jnp.bfloat16)[
        :, jnp.asarray(_COL_PERM, dtype=jnp.int32)]
    feat = jax.lax.bitcast_convert_type(
        feat.reshape(H * W, C // 2, 2), jnp.int32)
    roi32 = roi.astype(jnp.int32).reshape(N_ROI * 4)
    tab = jnp.asarray(_QDIV_TAB, dtype=jnp.int32)
    out = _rpn_pool_sc(feat, roi32, tab)
    zero = features[0, 0, 0, 0] * jnp.float32(0.0)
    return out.reshape(N_ROI, POOL, POOL, C) + zero


# speedup vs baseline: 1.0116x; 1.0116x over previous
import functools

import jax
import jax.numpy as jnp
from jax import lax
from jax.experimental import pallas as pl
from jax.experimental.pallas import tpu as pltpu
from jax.experimental.pallas import tpu_sc as plsc

_ILV = plsc.PackFormat.INTERLEAVED

POOL = 7
H = 64
W = 64
C = 256
N_ROI = 2000
ROWS = N_ROI * POOL * POOL
GROUP = 32
NCHUNKS = -(-ROWS // GROUP)
ROWS_PAD = NCHUNKS * GROUP
NC, NS = 2, 16
NW = NC * NS
CHUNKS_PER = -(-NCHUNKS // NW)
NBUF = 3


def _rpn_pool_body(feat_hbm, roi_hbm, tab_hbm, out_hbm, roi_v, tab_v, idx_b,
                   w_b, g_b, ob_b, semg, semo):
    wid = lax.axis_index("s") * NC + lax.axis_index("c")
    pltpu.sync_copy(roi_hbm, roi_v)
    pltpu.sync_copy(tab_hbm, tab_v)
    start = wid * CHUNKS_PER
    count = jnp.minimum(CHUNKS_PER, NCHUNKS - start)
    lanes = lax.iota(jnp.int32, 16)
    l4 = lanes * 4

    def issue_gather(k, b):
        for g2 in range(2):
            p = (start + k) * GROUP + g2 * 16 + lanes
            p = jnp.minimum(p, ROWS - 1)
            n = p // (POOL * POOL)
            q = p - n * (POOL * POOL)
            i = plsc.load_gather(tab_v, [q])
            j = q - i * POOL
            b4 = n * 4
            y1 = plsc.load_gather(roi_v, [b4])
            x1 = plsc.load_gather(roi_v, [b4 + 1])
            y2 = plsc.load_gather(roi_v, [b4 + 2])
            x2 = plsc.load_gather(roi_v, [b4 + 3])
            h = jnp.maximum(x2 - x1, 1)
            w = jnp.maximum(y2 - y1, 1)
            h7 = h.astype(jnp.float32) / float(POOL)
            w7 = w.astype(jnp.float32) / float(POOL)
            rpos = i.astype(jnp.float32) * h7
            cpos = j.astype(jnp.float32) * w7
            r0 = rpos.astype(jnp.int32)
            c0 = cpos.astype(jnp.int32)
            rf = rpos - r0.astype(jnp.float32)
            cf = cpos - c0.astype(jnp.float32)
            r1 = jnp.minimum(r0 + 1, h - 1)
            c1 = jnp.minimum(c0 + 1, w - 1)
            R0 = jnp.clip(x1 + r0, 0, H - 1)
            R1 = jnp.clip(x1 + r1, 0, H - 1)
            C0 = jnp.clip(y1 + c0, 0, W - 1)
            C1 = jnp.clip(y1 + c1, 0, W - 1)
            lg = l4 + g2 * 64
            plsc.store_scatter(idx_b[b], [lg], R0 * W + C0)
            plsc.store_scatter(idx_b[b], [lg + 1], R0 * W + C1)
            plsc.store_scatter(idx_b[b], [lg + 2], R1 * W + C0)
            plsc.store_scatter(idx_b[b], [lg + 3], R1 * W + C1)
            wr0 = 1.0 - rf
            wc0 = 1.0 - cf
            plsc.store_scatter(w_b[b], [lg], wr0 * wc0)
            plsc.store_scatter(w_b[b], [lg + 1], wr0 * cf)
            plsc.store_scatter(w_b[b], [lg + 2], rf * wc0)
            plsc.store_scatter(w_b[b], [lg + 3], rf * cf)
        pltpu.async_copy(feat_hbm.at[idx_b[b]], g_b[b], semg[b])

    def process(k, b):
        pltpu.make_async_copy(feat_hbm.at[idx_b[b]], g_b[b], semg[b]).wait()

        @pl.when(k + 2 < count)
        def _pref():
            issue_gather(k + 2, (b + 2) % NBUF)

        @pl.when(k >= NBUF)
        def _drain():
            pltpu.make_async_copy(
                ob_b[b],
                out_hbm.at[pl.ds((start + k - NBUF) * GROUP * C, GROUP * C)],
                semo[b]).wait()

        g_v = g_b[b]
        ob_v = ob_b[b]
        w_v = w_b[b]

        def row_body(r, rcarry):
            r4 = r * 4
            w00 = plsc.load_gather(w_v, [jnp.full((16,), 0, jnp.int32) + r4])
            w01 = plsc.load_gather(w_v, [jnp.full((16,), 1, jnp.int32) + r4])
            w10 = plsc.load_gather(w_v, [jnp.full((16,), 2, jnp.int32) + r4])
            w11 = plsc.load_gather(w_v, [jnp.full((16,), 3, jnp.int32) + r4])
            for t in range(C // 32):
                s = pl.ds(t * 16, 16)
                a0, b0 = plsc.unpack(
                    plsc.bitcast(g_v[r4, s], jnp.bfloat16), format=_ILV)
                a1, b1 = plsc.unpack(
                    plsc.bitcast(g_v[r4 + 1, s], jnp.bfloat16), format=_ILV)
                a2, b2 = plsc.unpack(
                    plsc.bitcast(g_v[r4 + 2, s], jnp.bfloat16), format=_ILV)
                a3, b3 = plsc.unpack(
                    plsc.bitcast(g_v[r4 + 3, s], jnp.bfloat16), format=_ILV)
                acca = (a0 * w00 + a1 * w01) + (a2 * w10 + a3 * w11)
                accb = (b0 * w00 + b1 * w01) + (b2 * w10 + b3 * w11)
                ob_v[pl.ds(r * C + t * 16, 16)] = acca
                ob_v[pl.ds(r * C + C // 2 + t * 16, 16)] = accb
            return rcarry

        lax.fori_loop(0, GROUP, row_body, 0, unroll=2)
        pltpu.async_copy(
            ob_v, out_hbm.at[pl.ds((start + k) * GROUP * C, GROUP * C)],
            semo[b])

    for pb in range(NBUF - 1):
        @pl.when(pb < count)
        def _prime():
            issue_gather(pb, pb)

    def outer(t, carry):
        for b in range(NBUF):
            k = t * NBUF + b

            @pl.when(k < count)
            def _():
                process(k, b)

        return carry

    lax.fori_loop(0, CHUNKS_PER // NBUF, outer, 0)

    for b in range(NBUF):
        last = count - 1 - ((count - 1 - b) % NBUF)
        pltpu.make_async_copy(
            ob_b[b], out_hbm.at[pl.ds((start + last) * GROUP * C, GROUP * C)],
            semo[b]).wait()


@functools.partial(
    pl.kernel,
    out_type=jax.ShapeDtypeStruct((ROWS_PAD * C,), jnp.float32),
    mesh=plsc.VectorSubcoreMesh(core_axis_name="c", subcore_axis_name="s"),
    scratch_types=[
        pltpu.VMEM((N_ROI * 4,), jnp.int32),
        pltpu.VMEM((64,), jnp.int32),
        [pltpu.VMEM((GROUP * 4,), jnp.int32) for _ in range(NBUF)],
        [pltpu.VMEM((GROUP * 4,), jnp.float32) for _ in range(NBUF)],
        [pltpu.VMEM((GROUP * 4, C // 2), jnp.int32) for _ in range(NBUF)],
        [pltpu.VMEM((GROUP * C,), jnp.float32) for _ in range(NBUF)],
        [pltpu.SemaphoreType.DMA for _ in range(NBUF)],
        [pltpu.SemaphoreType.DMA for _ in range(NBUF)],
    ],
    compiler_params=pltpu.CompilerParams(needs_layout_passes=False),
)
def _rpn_pool_sc(feat_hbm, roi_hbm, tab_hbm, out_hbm, roi_v, tab_v, idx_b,
                 w_b, g_b, ob_b, semg, semo):
    _rpn_pool_body(feat_hbm, roi_hbm, tab_hbm, out_hbm, roi_v, tab_v, idx_b,
                   w_b, g_b, ob_b, semg, semo)


_QDIV_TAB = [min(e, POOL * POOL - 1) // POOL for e in range(64)]

_COL_PERM = [(e // 2) + (e % 2) * (C // 2) for e in range(C)]


def kernel(features, roi):
    feat = features.reshape(H * W, C).astype(jnp.bfloat16)[
        :, jnp.asarray(_COL_PERM, dtype=jnp.int32)]
    feat = jax.lax.bitcast_convert_type(
        feat.reshape(H * W, C // 2, 2), jnp.int32)
    roi32 = roi.astype(jnp.int32).reshape(N_ROI * 4)
    tab = jnp.asarray(_QDIV_TAB, dtype=jnp.int32)
    out = _rpn_pool_sc(feat, roi32, tab)
    return out[:ROWS * C].reshape(N_ROI, POOL, POOL, C)


# speedup vs baseline: 1.0766x; 1.0643x over previous
import functools

import jax
import jax.numpy as jnp
from jax import lax
from jax.experimental import pallas as pl
from jax.experimental.pallas import tpu as pltpu
from jax.experimental.pallas import tpu_sc as plsc

_ILV = plsc.PackFormat.INTERLEAVED

POOL = 7
H = 64
W = 64
C = 256
N_ROI = 2000
ROWS = N_ROI * POOL * POOL
GROUP = 16
NCHUNKS = ROWS // GROUP
NC, NS = 2, 16
NW = NC * NS
CHUNKS_PER = -(-NCHUNKS // NW)


def _rpn_pool_body(feat_hbm, roi_hbm, tab_hbm, out_hbm, roi_v, tab_v, idx0,
                   idx1, w0, w1, g0, g1, ob0, ob1, semg0, semg1, semo0,
                   semo1):
    idx_b = (idx0, idx1)
    w_b = (w0, w1)
    g_b = (g0, g1)
    ob_b = (ob0, ob1)
    semg = (semg0, semg1)
    semo = (semo0, semo1)

    wid = lax.axis_index("s") * NC + lax.axis_index("c")
    pltpu.sync_copy(roi_hbm, roi_v)
    start = wid * CHUNKS_PER
    count = jnp.minimum(CHUNKS_PER, NCHUNKS - start)
    lanes = lax.iota(jnp.int32, 16)
    l4 = lanes * 4

    pltpu.sync_copy(tab_hbm, tab_v)

    def issue_gather(k, b):
        p = (start + k) * GROUP + lanes
        n = p // (POOL * POOL)
        q = p - n * (POOL * POOL)
        i = plsc.load_gather(tab_v, [q])
        j = q - i * POOL
        b4 = n * 4
        y1 = plsc.load_gather(roi_v, [b4])
        x1 = plsc.load_gather(roi_v, [b4 + 1])
        y2 = plsc.load_gather(roi_v, [b4 + 2])
        x2 = plsc.load_gather(roi_v, [b4 + 3])
        h = jnp.maximum(x2 - x1, 1)
        w = jnp.maximum(y2 - y1, 1)
        h7 = h.astype(jnp.float32) / float(POOL)
        w7 = w.astype(jnp.float32) / float(POOL)
        rpos = i.astype(jnp.float32) * h7
        cpos = j.astype(jnp.float32) * w7
        r0 = rpos.astype(jnp.int32)
        c0 = cpos.astype(jnp.int32)
        rf = rpos - r0.astype(jnp.float32)
        cf = cpos - c0.astype(jnp.float32)
        r1 = jnp.minimum(r0 + 1, h - 1)
        c1 = jnp.minimum(c0 + 1, w - 1)
        R0 = jnp.clip(x1 + r0, 0, H - 1)
        R1 = jnp.clip(x1 + r1, 0, H - 1)
        C0 = jnp.clip(y1 + c0, 0, W - 1)
        C1 = jnp.clip(y1 + c1, 0, W - 1)
        plsc.store_scatter(idx_b[b], [l4], R0 * W + C0)
        plsc.store_scatter(idx_b[b], [l4 + 1], R0 * W + C1)
        plsc.store_scatter(idx_b[b], [l4 + 2], R1 * W + C0)
        plsc.store_scatter(idx_b[b], [l4 + 3], R1 * W + C1)
        wr0 = 1.0 - rf
        wc0 = 1.0 - cf
        plsc.store_scatter(w_b[b], [l4], wr0 * wc0)
        plsc.store_scatter(w_b[b], [l4 + 1], wr0 * cf)
        plsc.store_scatter(w_b[b], [l4 + 2], rf * wc0)
        plsc.store_scatter(w_b[b], [l4 + 3], rf * cf)
        pltpu.async_copy(feat_hbm.at[idx_b[b]], g_b[b], semg[b])

    def process(k, b):
        @pl.when(k + 1 < count)
        def _pref():
            issue_gather(k + 1, 1 - b)

        pltpu.make_async_copy(feat_hbm.at[idx_b[b]], g_b[b], semg[b]).wait()

        @pl.when(k >= 2)
        def _drain():
            pltpu.make_async_copy(
                ob_b[b],
                out_hbm.at[pl.ds((start + k - 2) * GROUP * C, GROUP * C)],
                semo[b]).wait()

        g_v = g_b[b]
        ob_v = ob_b[b]
        w_v = w_b[b]

        def row_body(r, rcarry):
            r4 = r * 4
            w00 = plsc.load_gather(w_v, [jnp.full((16,), 0, jnp.int32) + r4])
            w01 = plsc.load_gather(w_v, [jnp.full((16,), 1, jnp.int32) + r4])
            w10 = plsc.load_gather(w_v, [jnp.full((16,), 2, jnp.int32) + r4])
            w11 = plsc.load_gather(w_v, [jnp.full((16,), 3, jnp.int32) + r4])
            for t in range(C // 32):
                s = pl.ds(t * 16, 16)
                a0, b0 = plsc.unpack(
                    plsc.bitcast(g_v[r4, s], jnp.bfloat16), format=_ILV)
                a1, b1 = plsc.unpack(
                    plsc.bitcast(g_v[r4 + 1, s], jnp.bfloat16), format=_ILV)
                a2, b2 = plsc.unpack(
                    plsc.bitcast(g_v[r4 + 2, s], jnp.bfloat16), format=_ILV)
                a3, b3 = plsc.unpack(
                    plsc.bitcast(g_v[r4 + 3, s], jnp.bfloat16), format=_ILV)
                acca = (a0 * w00 + a1 * w01) + (a2 * w10 + a3 * w11)
                accb = (b0 * w00 + b1 * w01) + (b2 * w10 + b3 * w11)
                ob_v[pl.ds(r * C + t * 16, 16)] = acca
                ob_v[pl.ds(r * C + C // 2 + t * 16, 16)] = accb
            return rcarry

        lax.fori_loop(0, GROUP, row_body, 0, unroll=4)
        pltpu.async_copy(
            ob_v, out_hbm.at[pl.ds((start + k) * GROUP * C, GROUP * C)],
            semo[b])

    issue_gather(0, 0)

    def outer(t, carry):
        for b in range(2):
            k = t * 2 + b

            @pl.when(k < count)
            def _():
                process(k, b)

        return carry

    lax.fori_loop(0, CHUNKS_PER // 2, outer, 0)

    for b in range(2):
        last = count - 1 - ((count - 1 + b) % 2)
        pltpu.make_async_copy(
            ob_b[b], out_hbm.at[pl.ds((start + last) * GROUP * C, GROUP * C)],
            semo[b]).wait()


@functools.partial(
    pl.kernel,
    out_type=jax.ShapeDtypeStruct((ROWS * C,), jnp.float32),
    mesh=plsc.VectorSubcoreMesh(core_axis_name="c", subcore_axis_name="s"),
    scratch_types=[
        pltpu.VMEM((N_ROI * 4,), jnp.int32),
        pltpu.VMEM((64,), jnp.int32),
        pltpu.VMEM((GROUP * 4,), jnp.int32),
        pltpu.VMEM((GROUP * 4,), jnp.int32),
        pltpu.VMEM((GROUP * 4,), jnp.float32),
        pltpu.VMEM((GROUP * 4,), jnp.float32),
        pltpu.VMEM((GROUP * 4, C // 2), jnp.int32),
        pltpu.VMEM((GROUP * 4, C // 2), jnp.int32),
        pltpu.VMEM((GROUP * C,), jnp.float32),
        pltpu.VMEM((GROUP * C,), jnp.float32),
        pltpu.SemaphoreType.DMA,
        pltpu.SemaphoreType.DMA,
        pltpu.SemaphoreType.DMA,
        pltpu.SemaphoreType.DMA,
    ],
    compiler_params=pltpu.CompilerParams(needs_layout_passes=False),
)
def _rpn_pool_sc(feat_hbm, roi_hbm, tab_hbm, out_hbm, roi_v, tab_v, idx0,
                 idx1, w0, w1, g0, g1, ob0, ob1, semg0, semg1, semo0, semo1):
    _rpn_pool_body(feat_hbm, roi_hbm, tab_hbm, out_hbm, roi_v, tab_v, idx0,
                   idx1, w0, w1, g0, g1, ob0, ob1, semg0, semg1, semo0, semo1)


_QDIV_TAB = [min(e, POOL * POOL - 1) // POOL for e in range(64)]


_COL_PERM = [(e // 2) + (e % 2) * (C // 2) for e in range(C)]


def kernel(features, roi):
    feat = features.reshape(H * W, C).astype(jnp.bfloat16)[
        :, jnp.asarray(_COL_PERM, dtype=jnp.int32)]
    feat = jax.lax.bitcast_convert_type(
        feat.reshape(H * W, C // 2, 2), jnp.int32)
    roi32 = roi.astype(jnp.int32).reshape(N_ROI * 4)
    tab = jnp.asarray(_QDIV_TAB, dtype=jnp.int32)
    out = _rpn_pool_sc(feat, roi32, tab)
    return out.reshape(N_ROI, POOL, POOL, C)


# speedup vs baseline: 1.0879x; 1.0105x over previous
import functools

import jax
import jax.numpy as jnp
from jax import lax
from jax.experimental import pallas as pl
from jax.experimental.pallas import tpu as pltpu
from jax.experimental.pallas import tpu_sc as plsc

_ILV = plsc.PackFormat.INTERLEAVED

POOL = 7
H = 64
W = 64
C = 256
N_ROI = 2000
ROWS = N_ROI * POOL * POOL
GROUP = 16
NCHUNKS = ROWS // GROUP
NC, NS = 2, 16
NW = NC * NS
CHUNKS_PER = -(-NCHUNKS // NW)


def _rpn_pool_body(feat_hbm, roi_hbm, tab_hbm, out_hbm, roi_v, tab_v, idx0,
                   idx1, w0, w1, g0, g1, ob0, ob1, semg0, semg1, semo0,
                   semo1):
    idx_b = (idx0, idx1)
    w_b = (w0, w1)
    g_b = (g0, g1)
    ob_b = (ob0, ob1)
    semg = (semg0, semg1)
    semo = (semo0, semo1)

    wid = lax.axis_index("s") * NC + lax.axis_index("c")
    pltpu.sync_copy(roi_hbm, roi_v)
    start = wid * CHUNKS_PER
    count = jnp.minimum(CHUNKS_PER, NCHUNKS - start)
    lanes = lax.iota(jnp.int32, 16)
    l4 = lanes * 4

    pltpu.sync_copy(tab_hbm, tab_v)

    def issue_gather(k, b):
        p = (start + k) * GROUP + lanes
        n = p // (POOL * POOL)
        q = p - n * (POOL * POOL)
        i = plsc.load_gather(tab_v, [q])
        j = q - i * POOL
        b4 = n * 4
        y1 = plsc.load_gather(roi_v, [b4])
        x1 = plsc.load_gather(roi_v, [b4 + 1])
        y2 = plsc.load_gather(roi_v, [b4 + 2])
        x2 = plsc.load_gather(roi_v, [b4 + 3])
        h = jnp.maximum(x2 - x1, 1)
        w = jnp.maximum(y2 - y1, 1)
        h7 = h.astype(jnp.float32) / float(POOL)
        w7 = w.astype(jnp.float32) / float(POOL)
        rpos = i.astype(jnp.float32) * h7
        cpos = j.astype(jnp.float32) * w7
        r0 = rpos.astype(jnp.int32)
        c0 = cpos.astype(jnp.int32)
        rf = rpos - r0.astype(jnp.float32)
        cf = cpos - c0.astype(jnp.float32)
        r1 = jnp.minimum(r0 + 1, h - 1)
        c1 = jnp.minimum(c0 + 1, w - 1)
        R0 = jnp.clip(x1 + r0, 0, H - 1)
        R1 = jnp.clip(x1 + r1, 0, H - 1)
        C0 = jnp.clip(y1 + c0, 0, W - 1)
        C1 = jnp.clip(y1 + c1, 0, W - 1)
        plsc.store_scatter(idx_b[b], [l4], R0 * W + C0)
        plsc.store_scatter(idx_b[b], [l4 + 1], R0 * W + C1)
        plsc.store_scatter(idx_b[b], [l4 + 2], R1 * W + C0)
        plsc.store_scatter(idx_b[b], [l4 + 3], R1 * W + C1)
        wr0 = 1.0 - rf
        wc0 = 1.0 - cf
        plsc.store_scatter(w_b[b], [l4], wr0 * wc0)
        plsc.store_scatter(w_b[b], [l4 + 1], wr0 * cf)
        plsc.store_scatter(w_b[b], [l4 + 2], rf * wc0)
        plsc.store_scatter(w_b[b], [l4 + 3], rf * cf)
        pltpu.async_copy(feat_hbm.at[idx_b[b]], g_b[b], semg[b])

    def process(k, b):
        @pl.when(k + 1 < count)
        def _pref():
            issue_gather(k + 1, 1 - b)

        pltpu.make_async_copy(feat_hbm.at[idx_b[b]], g_b[b], semg[b]).wait()

        @pl.when(k >= 2)
        def _drain():
            pltpu.make_async_copy(
                ob_b[b],
                out_hbm.at[pl.ds((start + k - 2) * GROUP * C, GROUP * C)],
                semo[b]).wait()

        g_v = g_b[b]
        ob_v = ob_b[b]
        w_v = w_b[b]

        def row_body(r, rcarry):
            r4 = r * 4
            w00 = plsc.load_gather(w_v, [jnp.full((16,), 0, jnp.int32) + r4])
            w01 = plsc.load_gather(w_v, [jnp.full((16,), 1, jnp.int32) + r4])
            w10 = plsc.load_gather(w_v, [jnp.full((16,), 2, jnp.int32) + r4])
            w11 = plsc.load_gather(w_v, [jnp.full((16,), 3, jnp.int32) + r4])
            wb00 = plsc.pack(w00, w00, format=_ILV)
            wb01 = plsc.pack(w01, w01, format=_ILV)
            wb10 = plsc.pack(w10, w10, format=_ILV)
            wb11 = plsc.pack(w11, w11, format=_ILV)
            for t in range(C // 32):
                s = pl.ds(t * 16, 16)
                m0 = plsc.bitcast(g_v[r4, s], jnp.bfloat16) * wb00
                m1 = plsc.bitcast(g_v[r4 + 1, s], jnp.bfloat16) * wb01
                m2 = plsc.bitcast(g_v[r4 + 2, s], jnp.bfloat16) * wb10
                m3 = plsc.bitcast(g_v[r4 + 3, s], jnp.bfloat16) * wb11
                acca, accb = plsc.unpack((m0 + m1) + (m2 + m3), format=_ILV)
                ob_v[pl.ds(r * C + t * 16, 16)] = acca
                ob_v[pl.ds(r * C + C // 2 + t * 16, 16)] = accb
            return rcarry

        lax.fori_loop(0, GROUP, row_body, 0, unroll=4)
        pltpu.async_copy(
            ob_v, out_hbm.at[pl.ds((start + k) * GROUP * C, GROUP * C)],
            semo[b])

    issue_gather(0, 0)

    def outer(t, carry):
        for b in range(2):
            k = t * 2 + b

            @pl.when(k < count)
            def _():
                process(k, b)

        return carry

    lax.fori_loop(0, CHUNKS_PER // 2, outer, 0)

    for b in range(2):
        last = count - 1 - ((count - 1 + b) % 2)
        pltpu.make_async_copy(
            ob_b[b], out_hbm.at[pl.ds((start + last) * GROUP * C, GROUP * C)],
            semo[b]).wait()


@functools.partial(
    pl.kernel,
    out_type=jax.ShapeDtypeStruct((ROWS * C,), jnp.float32),
    mesh=plsc.VectorSubcoreMesh(core_axis_name="c", subcore_axis_name="s"),
    scratch_types=[
        pltpu.VMEM((N_ROI * 4,), jnp.int32),
        pltpu.VMEM((64,), jnp.int32),
        pltpu.VMEM((GROUP * 4,), jnp.int32),
        pltpu.VMEM((GROUP * 4,), jnp.int32),
        pltpu.VMEM((GROUP * 4,), jnp.float32),
        pltpu.VMEM((GROUP * 4,), jnp.float32),
        pltpu.VMEM((GROUP * 4, C // 2), jnp.int32),
        pltpu.VMEM((GROUP * 4, C // 2), jnp.int32),
        pltpu.VMEM((GROUP * C,), jnp.float32),
        pltpu.VMEM((GROUP * C,), jnp.float32),
        pltpu.SemaphoreType.DMA,
        pltpu.SemaphoreType.DMA,
        pltpu.SemaphoreType.DMA,
        pltpu.SemaphoreType.DMA,
    ],
    compiler_params=pltpu.CompilerParams(needs_layout_passes=False),
)
def _rpn_pool_sc(feat_hbm, roi_hbm, tab_hbm, out_hbm, roi_v, tab_v, idx0,
                 idx1, w0, w1, g0, g1, ob0, ob1, semg0, semg1, semo0, semo1):
    _rpn_pool_body(feat_hbm, roi_hbm, tab_hbm, out_hbm, roi_v, tab_v, idx0,
                   idx1, w0, w1, g0, g1, ob0, ob1, semg0, semg1, semo0, semo1)


_QDIV_TAB = [min(e, POOL * POOL - 1) // POOL for e in range(64)]


_COL_PERM = [(e // 2) + (e % 2) * (C // 2) for e in range(C)]


def kernel(features, roi):
    feat = features.reshape(H * W, C).astype(jnp.bfloat16)[
        :, jnp.asarray(_COL_PERM, dtype=jnp.int32)]
    feat = jax.lax.bitcast_convert_type(
        feat.reshape(H * W, C // 2, 2), jnp.int32)
    roi32 = roi.astype(jnp.int32).reshape(N_ROI * 4)
    tab = jnp.asarray(_QDIV_TAB, dtype=jnp.int32)
    out = _rpn_pool_sc(feat, roi32, tab)
    return out.reshape(N_ROI, POOL, POOL, C)
